# baseline (speedup 1.0000x reference)
import functools

import jax
import jax.numpy as jnp
from jax import lax
from jax.experimental import pallas as pl
from jax.experimental.pallas import tpu as pltpu
from jax.experimental.pallas import tpu_sc as plsc

N2 = 8192
K = 10
R2V = 0.0625
L = 16
NW = 32
QPW = N2 // NW
B = 4


def _sc_body(xs_h, ys_h, zs_h, qx_h, qy_h, qz_h,
             map_h, o3_h,
             xs, ys, zs, qx, qy, qz, mbuf, o3b):
    wid = lax.axis_index("s") * 2 + lax.axis_index("c")
    base = wid * QPW

    pltpu.sync_copy(xs_h, xs)
    pltpu.sync_copy(ys_h, ys)
    pltpu.sync_copy(zs_h, zs)
    pltpu.sync_copy(qx_h.at[pl.ds(base, QPW)], qx)
    pltpu.sync_copy(qy_h.at[pl.ds(base, QPW)], qy)
    pltpu.sync_copy(qz_h.at[pl.ds(base, QPW)], qz)

    zi = jnp.zeros((L,), jnp.int32)
    flat_mbuf = QPW * K

    def zero_body(i, _):
        mbuf[pl.ds(i * L, L)] = zi
        return 0

    lax.fori_loop(0, flat_mbuf // L, zero_body, 0)

    iota = lax.iota(jnp.int32, L)
    iota_c9 = jnp.minimum(iota, K - 1)
    m10 = iota < K

    def qblock_body(qb, _):
        qvx = qx[pl.ds(qb * L, L)]
        qvy = qy[pl.ds(qb * L, L)]
        qvz = qz[pl.ds(qb * L, L)]
        for batch in range(L // B):
            qs = [(qvx[batch * B + b], qvy[batch * B + b], qvz[batch * B + b])
                  for b in range(B)]

            def cond(carry):
                j = carry[0]
                cnts = carry[1:]
                not_done = cnts[0] < K
                for c in cnts[1:]:
                    not_done = jnp.logical_or(not_done, c < K)
                return jnp.logical_and(j < N2, not_done)

            def step(carry, qs=qs, batch=batch):
                j = carry[0]
                cnts = list(carry[1:])
                cx = xs[pl.ds(j, L)]
                cy = ys[pl.ds(j, L)]
                cz = zs[pl.ds(j, L)]
                cand = j + iota
                for b in range(B):
                    qxs, qys, qzs = qs[b]
                    q = qb * L + batch * B + b
                    dx = cx - qxs
                    dy = cy - qys
                    dz = cz - qzs
                    d2 = dx * dx + dy * dy + dz * dz
                    within = d2 <= R2V
                    wi = within.astype(jnp.int32)
                    excl = plsc.cumsum(wi) - wi
                    n_b = plsc.all_reduce_population_count(within)
                    slot = excl + cnts[b]
                    valid = jnp.logical_and(within, slot < K)
                    plsc.store_scatter(mbuf, [q * K + slot], cand, mask=valid)
                    cnts[b] = cnts[b] + n_b[0]
                return (j + L, *cnts)

            final = lax.while_loop(cond, step,
                                   (jnp.int32(0),) + (jnp.int32(0),) * B)
            for b in range(B):
                q = qb * L + batch * B + b
                row = mbuf[pl.ds(q * K, L)]
                vmask = iota < jnp.minimum(final[1 + b], K)
                gx = plsc.load_gather(xs, [row])
                gy = plsc.load_gather(ys, [row])
                gz = plsc.load_gather(zs, [row])
                qsplat = jnp.full((L,), q, jnp.int32)
                for d, g in enumerate((gx, gy, gz)):
                    plsc.store_scatter(
                        o3b, [qsplat, iota_c9, jnp.full((L,), d, jnp.int32)],
                        jnp.where(vmask, g, 0.0), mask=m10)
        return 0

    lax.fori_loop(0, QPW // L, qblock_body, 0)

    pltpu.sync_copy(mbuf.at[pl.ds(0, flat_mbuf)],
                    map_h.at[pl.ds(base * K, flat_mbuf)])
    pltpu.sync_copy(o3b, o3_h.at[pl.ds(base, QPW)])


_sc_ball_query = functools.partial(
    pl.kernel,
    out_type=[
        jax.ShapeDtypeStruct((N2 * K,), jnp.int32),
        jax.ShapeDtypeStruct((N2, K, 3), jnp.float32),
    ],
    mesh=plsc.VectorSubcoreMesh(core_axis_name="c", subcore_axis_name="s"),
    compiler_params=pltpu.CompilerParams(
        needs_layout_passes=False, use_tc_tiling_on_sc=False),
    scratch_types=[
        pltpu.VMEM((N2,), jnp.float32),
        pltpu.VMEM((N2,), jnp.float32),
        pltpu.VMEM((N2,), jnp.float32),
        pltpu.VMEM((QPW,), jnp.float32),
        pltpu.VMEM((QPW,), jnp.float32),
        pltpu.VMEM((QPW,), jnp.float32),
        pltpu.VMEM((QPW * K + L,), jnp.int32),
        pltpu.VMEM((QPW, K, 3), jnp.float32),
    ],
)(_sc_body)


@jax.jit
def kernel(x, p_grid):
    b = x.shape[0]
    x2 = x[0]
    p2 = p_grid.reshape(N2, 3)
    m, o3 = _sc_ball_query(
        x2[:, 0], x2[:, 1], x2[:, 2], p2[:, 0], p2[:, 1], p2[:, 2])
    return m.reshape(b, N2, K), o3.reshape(b, N2, K, 3)


# speedup vs baseline: 1.8337x; 1.8337x over previous
import functools

import jax
import jax.numpy as jnp
from jax import lax
from jax.experimental import pallas as pl
from jax.experimental.pallas import tpu as pltpu
from jax.experimental.pallas import tpu_sc as plsc

N2 = 8192
K = 10
R2V = 0.0625
L = 16
NW = 32
QPW = N2 // NW
ROW = 16
B = 4


def _sc_body(xs_h, ys_h, zs_h, qx_h, qy_h, qz_h,
             map_h, ox_h, oy_h, oz_h,
             xs, ys, zs, qx, qy, qz, mbuf, oxb, oyb, ozb):
    wid = lax.axis_index("s") * 2 + lax.axis_index("c")
    base = wid * QPW

    pltpu.sync_copy(xs_h, xs)
    pltpu.sync_copy(ys_h, ys)
    pltpu.sync_copy(zs_h, zs)
    pltpu.sync_copy(qx_h.at[pl.ds(base, QPW)], qx)
    pltpu.sync_copy(qy_h.at[pl.ds(base, QPW)], qy)
    pltpu.sync_copy(qz_h.at[pl.ds(base, QPW)], qz)

    zi = jnp.zeros((L,), jnp.int32)
    zf = jnp.zeros((L,), jnp.float32)

    def zero_body(i, _):
        mbuf[pl.ds(i * L, L)] = zi
        oxb[pl.ds(i * L, L)] = zf
        oyb[pl.ds(i * L, L)] = zf
        ozb[pl.ds(i * L, L)] = zf
        return 0

    lax.fori_loop(0, QPW, zero_body, 0)

    iota = lax.iota(jnp.int32, L)

    def qblock_body(qb, _):
        qvx = qx[pl.ds(qb * L, L)]
        qvy = qy[pl.ds(qb * L, L)]
        qvz = qz[pl.ds(qb * L, L)]
        for batch in range(L // B):
            qs = [(qvx[batch * B + b], qvy[batch * B + b], qvz[batch * B + b])
                  for b in range(B)]

            def cond(carry):
                j = carry[0]
                cnts = carry[1:]
                not_done = cnts[0] < K
                for c in cnts[1:]:
                    not_done = jnp.logical_or(not_done, c < K)
                return jnp.logical_and(j < N2, not_done)

            def step(carry, qs=qs, batch=batch):
                j = carry[0]
                cnts = list(carry[1:])
                cx = xs[pl.ds(j, L)]
                cy = ys[pl.ds(j, L)]
                cz = zs[pl.ds(j, L)]
                cand = j + iota
                for b in range(B):
                    qxs, qys, qzs = qs[b]
                    q = qb * L + batch * B + b
                    dx = cx - qxs
                    dy = cy - qys
                    dz = cz - qzs
                    d2 = dx * dx + dy * dy + dz * dz
                    within = d2 <= R2V
                    wi = within.astype(jnp.int32)
                    excl = plsc.cumsum(wi) - wi
                    n_b = plsc.all_reduce_population_count(within)
                    slot = excl + cnts[b]
                    valid = jnp.logical_and(within, slot < K)
                    fidx = q * ROW + slot
                    plsc.store_scatter(mbuf, [fidx], cand, mask=valid)
                    plsc.store_scatter(oxb, [fidx], cx, mask=valid)
                    plsc.store_scatter(oyb, [fidx], cy, mask=valid)
                    plsc.store_scatter(ozb, [fidx], cz, mask=valid)
                    cnts[b] = cnts[b] + n_b[0]
                return (j + L, *cnts)

            lax.while_loop(cond, step,
                           (jnp.int32(0),) + (jnp.int32(0),) * B)
        return 0

    lax.fori_loop(0, QPW // L, qblock_body, 0)

    pltpu.sync_copy(mbuf, map_h.at[pl.ds(base * ROW, QPW * ROW)])
    pltpu.sync_copy(oxb, ox_h.at[pl.ds(base * ROW, QPW * ROW)])
    pltpu.sync_copy(oyb, oy_h.at[pl.ds(base * ROW, QPW * ROW)])
    pltpu.sync_copy(ozb, oz_h.at[pl.ds(base * ROW, QPW * ROW)])


_sc_ball_query = functools.partial(
    pl.kernel,
    out_type=[
        jax.ShapeDtypeStruct((N2 * ROW,), jnp.int32),
        jax.ShapeDtypeStruct((N2 * ROW,), jnp.float32),
        jax.ShapeDtypeStruct((N2 * ROW,), jnp.float32),
        jax.ShapeDtypeStruct((N2 * ROW,), jnp.float32),
    ],
    mesh=plsc.VectorSubcoreMesh(core_axis_name="c", subcore_axis_name="s"),
    compiler_params=pltpu.CompilerParams(needs_layout_passes=False),
    scratch_types=[
        pltpu.VMEM((N2,), jnp.float32),
        pltpu.VMEM((N2,), jnp.float32),
        pltpu.VMEM((N2,), jnp.float32),
        pltpu.VMEM((QPW,), jnp.float32),
        pltpu.VMEM((QPW,), jnp.float32),
        pltpu.VMEM((QPW,), jnp.float32),
        pltpu.VMEM((QPW * ROW,), jnp.int32),
        pltpu.VMEM((QPW * ROW,), jnp.float32),
        pltpu.VMEM((QPW * ROW,), jnp.float32),
        pltpu.VMEM((QPW * ROW,), jnp.float32),
    ],
)(_sc_body)


@jax.jit
def kernel(x, p_grid):
    b = x.shape[0]
    x2 = x[0]
    p2 = p_grid.reshape(N2, 3)
    m, ox, oy, oz = _sc_ball_query(
        x2[:, 0], x2[:, 1], x2[:, 2], p2[:, 0], p2[:, 1], p2[:, 2])
    mapping = m.reshape(N2, ROW)[:, :K]
    outputs = jnp.stack(
        [ox.reshape(N2, ROW)[:, :K],
         oy.reshape(N2, ROW)[:, :K],
         oz.reshape(N2, ROW)[:, :K]], axis=-1)
    return mapping.reshape(b, N2, K), outputs.reshape(b, N2, K, 3)
